# baseline (speedup 1.0000x reference)
import math

import jax
import jax.numpy as jnp
from jax import lax
from jax.experimental import pallas as pl
from jax.experimental.pallas import tpu as pltpu
from jax.experimental.pallas import tpu_sc as plsc

D = 64
L = 16
NC, NS = 2, 16
NW = NC * NS
R, S = 16384, 50
RPW = R // NW
P = 8
CH = P * S
NCHUNK = RPW // P
SCALE = math.sqrt(D)


def _emb_body(table_hbm, tok_hbm, out_hbm,
              idx0, idx1, rows0, rows1, blk, sem0, sem1):
    wid = lax.axis_index("s") * NC + lax.axis_index("c")
    rbase = wid * RPW
    fbase = rbase * S
    idx = (idx0, idx1)
    rows = (rows0, rows1)
    sems = (sem0, sem1)

    for b in range(2):
        pltpu.sync_copy(tok_hbm.at[pl.ds(fbase + b * CH, CH)], idx[b])
        pltpu.async_copy(table_hbm.at[idx[b]], rows[b], sems[b])

    @pl.loop(0, NCHUNK, step=2)
    def _chunks(g):
        for b in range(2):
            gb = g + b
            pltpu.make_async_copy(
                table_hbm.at[idx[b]], rows[b], sems[b]).wait()

            @plsc.parallel_loop(0, P, 1)
            def _scale_row(p):
                @pl.loop(0, S, unroll=5)
                def _scale_tok(s):
                    r = p * S + s
                    for j in range(D // L):
                        sl = pl.ds(j * L, L)
                        blk[p, s, sl] = rows[b][r, sl] * SCALE

            pltpu.sync_copy(blk, out_hbm.at[pl.ds(rbase + gb * P, P)])

            @pl.when(gb + 2 < NCHUNK)
            def _fire():
                nxt = fbase + (gb + 2) * CH
                pltpu.sync_copy(tok_hbm.at[pl.ds(nxt, CH)], idx[b])
                pltpu.async_copy(table_hbm.at[idx[b]], rows[b], sems[b])


def kernel(tokens, table):
    tok_flat = tokens.reshape(-1)
    mesh = plsc.VectorSubcoreMesh(core_axis_name="c", subcore_axis_name="s")
    k = pl.kernel(
        _emb_body,
        out_type=jax.ShapeDtypeStruct((R, S, D), jnp.float32),
        mesh=mesh,
        scratch_types=[
            pltpu.VMEM((CH,), jnp.int32),
            pltpu.VMEM((CH,), jnp.int32),
            pltpu.VMEM((CH, D), jnp.float32),
            pltpu.VMEM((CH, D), jnp.float32),
            pltpu.VMEM((P, S, D), jnp.float32),
            pltpu.SemaphoreType.DMA,
            pltpu.SemaphoreType.DMA,
        ],
        compiler_params=pltpu.CompilerParams(use_tc_tiling_on_sc=False),
    )
    return k(table, tok_flat)
